# Initial kernel scaffold; baseline (speedup 1.0000x reference)
import jax
import jax.numpy as jnp
from jax.experimental import pallas as pl


def kernel(z, emb_weight):
    raise NotImplementedError("write your pallas kernel here")


# speedup vs baseline: 1.2699x; 1.2699x over previous
import functools

import jax
import jax.numpy as jnp
from jax.experimental import pallas as pl

_VOCAB = 1024
_EMBED = 256
_TEMPERATURE = 0.5
_LOG_PARAM_Q = -2.995732273553991
_ROW_TILE = 1024


def _vq_kernel(z_ref, emb_ref, g_ref, tokens_ref, zq_ref):
    kappa = jnp.exp(jnp.float32(_LOG_PARAM_Q)) + 1.0

    e = emb_ref[...]
    en = e / jnp.maximum(
        jnp.sqrt(jnp.sum(e * e, axis=1, keepdims=True)), 1e-12)

    z = z_ref[...]
    zn = z / jnp.maximum(
        jnp.sqrt(jnp.sum(z * z, axis=1, keepdims=True)), 1e-12)

    logit = kappa * jax.lax.dot_general(
        zn, en, (((1,), (1,)), ((), ())),
        preferred_element_type=jnp.float32)

    s = (logit + g_ref[...]) * (1.0 / _TEMPERATURE)
    m = jnp.max(s, axis=1, keepdims=True)
    p = jnp.exp(s - m)
    tokens = p / jnp.sum(p, axis=1, keepdims=True)
    tokens_ref[...] = tokens

    zq_ref[...] = jax.lax.dot_general(
        tokens, en, (((1,), (0,)), ((), ())),
        preferred_element_type=jnp.float32)


@functools.partial(jax.jit, static_argnames=())
def kernel(z, emb_weight):
    n, d = z.shape
    k = emb_weight.shape[0]

    eps = 1e-10
    u = jax.random.uniform(jax.random.key(42), (n, k), dtype=jnp.float32)
    g = -jnp.log(-jnp.log(u + eps) + eps)

    grid = (n // _ROW_TILE,)
    tokens, zq = pl.pallas_call(
        _vq_kernel,
        grid=grid,
        in_specs=[
            pl.BlockSpec((_ROW_TILE, d), lambda i: (i, 0)),
            pl.BlockSpec((k, d), lambda i: (0, 0)),
            pl.BlockSpec((_ROW_TILE, k), lambda i: (i, 0)),
        ],
        out_specs=[
            pl.BlockSpec((_ROW_TILE, k), lambda i: (i, 0)),
            pl.BlockSpec((_ROW_TILE, d), lambda i: (i, 0)),
        ],
        out_shape=[
            jax.ShapeDtypeStruct((n, k), jnp.float32),
            jax.ShapeDtypeStruct((n, d), jnp.float32),
        ],
    )(z, emb_weight, g)
    return tokens, zq
